# baseline (speedup 1.0000x reference)
import jax
import jax.numpy as jnp
from jax.experimental import pallas as pl
from jax.experimental.pallas import tpu as pltpu

B, C, Hh, Ww = 8, 384, 32, 32
D = 64
K = 1024
HP = Hh + 2
ROWS = B * HP * HP
PAD0 = 48
EXT = 9344
VQC = 8
VQR = EXT // VQC
CC = 4
CR = ROWS // CC
SHIFTS = [(dh - 1) * HP + (dw - 1) for dh in range(3) for dw in range(3)]
f32 = jnp.float32
bf16 = jnp.bfloat16


def _conv9_chunked(x_ref, w_ref, b_row, out_ref, relu, mask_ref, nout):
    for c in range(CC):
        base = PAD0 + c * CR
        acc = None
        for k, s in enumerate(SHIFTS):
            x = x_ref[base + s:base + s + CR, :].astype(bf16)
            p = jax.lax.dot_general(x, w_ref[k], (((1,), (0,)), ((), ())),
                                    preferred_element_type=f32)
            acc = p if acc is None else acc + p
        acc = acc + b_row
        if relu:
            acc = jnp.maximum(acc, 0.0)
        out_ref[base:base + CR, :] = acc * mask_ref[base:base + CR, :]


def _enc_kernel(h_ref, wq1_ref, bq1_ref, wq2_ref, bq2_ref, cb_ref, mask_ref,
                zq_ref, idx_ref, loss_ref, z1_ref, ze_ref):
    z1_ref[...] = jnp.zeros((EXT, D), f32)
    ze_ref[...] = jnp.zeros((EXT, D), f32)
    _conv9_chunked(h_ref, wq1_ref, bq1_ref[0:1, :], z1_ref, True, mask_ref, D)
    _conv9_chunked(z1_ref, wq2_ref, bq2_ref[0:1, :], ze_ref, False, mask_ref, D)

    cb = cb_ref[...]
    cb_b = cb.astype(bf16)
    cnorm = jnp.sum(cb * cb, axis=1, keepdims=True).reshape(1, K)
    acc_loss = jnp.zeros((1, 1), f32)
    for c in range(VQC):
        z = ze_ref[c * VQR:(c + 1) * VQR, :]
        m = jax.lax.dot_general(z.astype(bf16), cb_b, (((1,), (1,)), ((), ())),
                                preferred_element_type=f32)
        znorm = jnp.sum(z * z, axis=1, keepdims=True)
        dist = (znorm - 2.0 * m) + cnorm
        minv = jnp.min(dist, axis=1, keepdims=True)
        iot = jax.lax.broadcasted_iota(jnp.int32, (VQR, K), 1)
        idx = jnp.min(jnp.where(dist == minv, iot, K), axis=1, keepdims=True)
        idx_ref[c * VQR:(c + 1) * VQR, :] = idx
        onehot = (iot == idx).astype(f32)
        zq = jax.lax.dot_general(onehot, cb, (((1,), (0,)), ((), ())),
                                 preferred_element_type=f32,
                                 precision=jax.lax.Precision.HIGHEST)
        zq = zq * mask_ref[c * VQR:(c + 1) * VQR, :]
        zq_ref[c * VQR:(c + 1) * VQR, :] = zq
        diff = z - zq
        acc_loss = acc_loss + jnp.sum(diff * diff).reshape(1, 1)
    loss_ref[...] = acc_loss * (1.0 / (B * Hh * Ww * D))


def _dec_kernel(zq_ref, wr1_ref, br1_ref, wr2_ref, br2_ref, wsk_ref, bsk_ref,
                mask_ref, hhat_ref, r1_ref):
    r1_ref[...] = jnp.zeros((EXT, C), f32)
    _conv9_chunked(zq_ref, wr1_ref, br1_ref[0:1, :], r1_ref, True, mask_ref, C)
    for c in range(CC):
        base = PAD0 + c * CR
        acc = None
        for k, s in enumerate(SHIFTS):
            x = r1_ref[base + s:base + s + CR, :].astype(bf16)
            p = jax.lax.dot_general(x, wr2_ref[k], (((1,), (0,)), ((), ())),
                                    preferred_element_type=f32)
            acc = p if acc is None else acc + p
        ysk = jax.lax.dot_general(zq_ref[base:base + CR, :].astype(bf16), wsk_ref[...],
                                  (((1,), (0,)), ((), ())),
                                  preferred_element_type=f32)
        hhat_ref[c * CR:(c + 1) * CR, :] = (acc + br2_ref[0:1, :]) + (ysk + bsk_ref[0:1, :])


def kernel(h, Wq1, bq1, Wq2, bq2, codebook, Wr1, br1, Wr2, br2, Wskip, bskip):
    hp = jnp.pad(jnp.transpose(h, (0, 2, 3, 1)), ((0, 0), (1, 1), (1, 1), (0, 0)))
    hflat = hp.reshape(ROWS, C)
    h_ext = jnp.pad(hflat, ((PAD0, EXT - PAD0 - ROWS), (0, 0)))

    wq1 = jnp.transpose(Wq1, (2, 3, 1, 0)).reshape(9, C, D).astype(bf16)
    wq2 = jnp.transpose(Wq2, (2, 3, 1, 0)).reshape(9, D, D).astype(bf16)
    wr1 = jnp.transpose(Wr1, (2, 3, 1, 0)).reshape(9, D, C).astype(bf16)
    wr2 = jnp.transpose(Wr2, (2, 3, 1, 0)).reshape(9, C, C).astype(bf16)
    wsk = jnp.transpose(Wskip, (2, 3, 1, 0)).reshape(D, C).astype(bf16)

    r = jnp.arange(EXT) - PAD0
    j = r % (HP * HP) % HP
    i = r % (HP * HP) // HP
    valid = (r >= 0) & (r < ROWS) & (i >= 1) & (i <= Hh) & (j >= 1) & (j <= Ww)
    mask = valid.astype(f32)[:, None]

    zq_ext, idx_ext, loss = pl.pallas_call(
        _enc_kernel,
        out_shape=(
            jax.ShapeDtypeStruct((EXT, D), f32),
            jax.ShapeDtypeStruct((EXT, 1), jnp.int32),
            jax.ShapeDtypeStruct((1, 1), f32),
        ),
        scratch_shapes=[
            pltpu.VMEM((EXT, D), f32),
            pltpu.VMEM((EXT, D), f32),
        ],
        compiler_params=pltpu.CompilerParams(
            vmem_limit_bytes=100 * 1024 * 1024,
        ),
    )(h_ext, wq1, bq1.reshape(1, D), wq2, bq2.reshape(1, D), codebook, mask)

    hhat_rows = pl.pallas_call(
        _dec_kernel,
        out_shape=jax.ShapeDtypeStruct((ROWS, C), f32),
        scratch_shapes=[pltpu.VMEM((EXT, C), f32)],
        compiler_params=pltpu.CompilerParams(
            vmem_limit_bytes=100 * 1024 * 1024,
        ),
    )(zq_ext, wr1, br1.reshape(1, C), wr2, br2.reshape(1, C), wsk,
      bskip.reshape(1, C), mask)

    zq = zq_ext[PAD0:PAD0 + ROWS].reshape(B, HP, HP, D)[:, 1:1 + Hh, 1:1 + Ww, :]
    z_q_st = jnp.transpose(zq, (0, 3, 1, 2))
    hh = hhat_rows.reshape(B, HP, HP, C)[:, 1:1 + Hh, 1:1 + Ww, :]
    h_hat = jnp.transpose(hh, (0, 3, 1, 2))
    indices = idx_ext[PAD0:PAD0 + ROWS, 0].reshape(B, HP, HP)[:, 1:1 + Hh, 1:1 + Ww]
    return (z_q_st, h_hat, loss.reshape(()), indices)
